# Initial kernel scaffold; baseline (speedup 1.0000x reference)
import jax
import jax.numpy as jnp
from jax.experimental import pallas as pl


def kernel(x, tables):
    raise NotImplementedError("write your pallas kernel here")


# speedup vs baseline: 3.1720x; 3.1720x over previous
import functools

import jax
import jax.numpy as jnp
from jax import lax
from jax.experimental import pallas as pl
from jax.experimental.pallas import tpu as pltpu
from jax.experimental.pallas import tpu_sc as plsc

B, L, F = 4096, 20, 26
VOCAB, DIM = 100000, 32
N = B * L * F
NC, NS = 2, 16
NW = NC * NS
PER_W = N // NW
IC = 128
IR = 8
CH = IR * IC
NCH = PER_W // CH
VG = IC // 16


def _build(ncores):
    mesh = plsc.VectorSubcoreMesh(core_axis_name="c", subcore_axis_name="s")

    @functools.partial(
        pl.kernel,
        mesh=mesh,
        out_type=jax.ShapeDtypeStruct((N, DIM), jnp.float32),
        scratch_types=[
            pltpu.VMEM((IR, IC), jnp.int32),
            pltpu.VMEM((CH, DIM), jnp.float32),
            pltpu.SemaphoreType.DMA,
        ],
        compiler_params=pltpu.CompilerParams(use_tc_tiling_on_sc=False),
    )
    def run(x_hbm, tab_hbm, out_hbm, idxb, rowsb, sem):
        wid = lax.axis_index("s") * ncores + lax.axis_index("c")
        base = wid * PER_W
        xrow0 = wid * (PER_W // IC)

        def chunk(c, carry):
            start = base + c * CH
            pltpu.sync_copy(x_hbm.at[pl.ds(xrow0 + c * IR, IR), :], idxb)

            def adj(i, cy):
                j = i // VG
                k = i % VG
                s = pl.ds(k * 16, 16)
                p = start + j * IC + k * 16 + lax.iota(jnp.int32, 16)
                idxb[j, s] = idxb[j, s] + (p % F) * VOCAB
                return cy

            lax.fori_loop(0, IR * VG, adj, 0)

            copies = [
                pltpu.async_copy(
                    tab_hbm.at[idxb.at[j]],
                    rowsb.at[pl.ds(j * IC, IC), :],
                    sem,
                )
                for j in range(IR)
            ]
            for cp in copies:
                cp.wait()
            pltpu.sync_copy(rowsb, out_hbm.at[pl.ds(start, CH), :])
            return carry

        lax.fori_loop(0, NCH, chunk, 0)

    return run


def kernel(x, tables):
    x2d = x.astype(jnp.int32).reshape(N // IC, IC)
    tab = tables.reshape(F * VOCAB, DIM)
    out = _build(NC)(x2d, tab)
    return out.reshape(B, L, F * DIM)


# speedup vs baseline: 3.2654x; 1.0295x over previous
import functools

import jax
import jax.numpy as jnp
from jax import lax
from jax.experimental import pallas as pl
from jax.experimental.pallas import tpu as pltpu
from jax.experimental.pallas import tpu_sc as plsc

B, L, F = 4096, 20, 26
VOCAB, DIM = 100000, 32
N = B * L * F
NC, NS = 2, 16
NW = NC * NS
PER_W = N // NW
IC = 128
IR = 8
CH = IR * IC
NCH = PER_W // CH
VG = IC // 16
NPAIR = (NCH - 1) // 2


def _build(ncores):
    mesh = plsc.VectorSubcoreMesh(core_axis_name="c", subcore_axis_name="s")

    @functools.partial(
        pl.kernel,
        mesh=mesh,
        out_type=jax.ShapeDtypeStruct((N, DIM), jnp.float32),
        scratch_types=[
            pltpu.VMEM((IR, IC), jnp.int32),
            pltpu.VMEM((IR, IC), jnp.int32),
            pltpu.VMEM((CH, DIM), jnp.float32),
            pltpu.VMEM((CH, DIM), jnp.float32),
            pltpu.SemaphoreType.DMA,
            pltpu.SemaphoreType.DMA,
            pltpu.SemaphoreType.DMA,
            pltpu.SemaphoreType.DMA,
        ],
        compiler_params=pltpu.CompilerParams(use_tc_tiling_on_sc=False),
    )
    def run(x_hbm, tab_hbm, out_hbm, idx0, idx1, rows0, rows1,
            gsem0, gsem1, osem0, osem1):
        idxb = (idx0, idx1)
        rowsb = (rows0, rows1)
        gsem = (gsem0, gsem1)
        osem = (osem0, osem1)

        wid = lax.axis_index("s") * ncores + lax.axis_index("c")
        base = wid * PER_W
        xrow0 = wid * (PER_W // IC)

        def load_adjust(c, b):
            pltpu.sync_copy(x_hbm.at[pl.ds(xrow0 + c * IR, IR), :], idxb[b])
            start = base + c * CH

            def adj(i, cy):
                j = i // VG
                k = i % VG
                s = pl.ds(k * 16, 16)
                p = start + j * IC + k * 16 + lax.iota(jnp.int32, 16)
                idxb[b][j, s] = idxb[b][j, s] + (p % F) * VOCAB
                return cy

            lax.fori_loop(0, IR * VG, adj, 0)

        def fire_gathers(b):
            for j in range(IR):
                pltpu.async_copy(
                    tab_hbm.at[idxb[b].at[j]],
                    rowsb[b].at[pl.ds(j * IC, IC), :],
                    gsem[b],
                )

        def wait_gathers(b):
            for j in range(IR):
                pltpu.make_async_copy(
                    tab_hbm.at[idxb[b].at[j]],
                    rowsb[b].at[pl.ds(j * IC, IC), :],
                    gsem[b],
                ).wait()

        def fire_writeback(c, b):
            pltpu.async_copy(rowsb[b], out_hbm.at[pl.ds(base + c * CH, CH), :],
                             osem[b])

        def wait_writeback(c, b):
            pltpu.make_async_copy(rowsb[b],
                                  out_hbm.at[pl.ds(base + c * CH, CH), :],
                                  osem[b]).wait()

        load_adjust(0, 0)
        fire_gathers(0)

        def pair(t, carry):
            c = 2 * t
            wait_gathers(0)
            fire_writeback(c, 0)
            load_adjust(c + 1, 1)

            @pl.when(t > 0)
            def _():
                wait_writeback(c - 1, 1)

            fire_gathers(1)
            wait_gathers(1)
            fire_writeback(c + 1, 1)
            load_adjust(c + 2, 0)
            wait_writeback(c, 0)
            fire_gathers(0)
            return carry

        lax.fori_loop(0, NPAIR, pair, 0)

        wait_gathers(0)
        wait_writeback(NCH - 2, 1)
        pltpu.sync_copy(rowsb[0], out_hbm.at[pl.ds(base + (NCH - 1) * CH, CH), :])

    return run


def kernel(x, tables):
    x2d = x.astype(jnp.int32).reshape(N // IC, IC)
    tab = tables.reshape(F * VOCAB, DIM)
    out = _build(NC)(x2d, tab)
    return out.reshape(B, L, F * DIM)
